# baseline (speedup 1.0000x reference)
import functools

import jax
import jax.numpy as jnp
from jax import lax
from jax.experimental import pallas as pl
from jax.experimental.pallas import tpu as pltpu
from jax.experimental.pallas import tpu_sc as plsc

N = 10000
E = 320000
H = 128
HC = H // 2
L = 4
NC = 2
NS = 16
EPW = E // NS
CH = 40
NCH = EPW // CH
IDXB = 20
NBLK = NCH // IDXB
ROWC = E // CH
ROWS_PER_TILE = 632
NP = ROWS_PER_TILE * NS

_PREC = lax.Precision.DEFAULT


def _dot(x, w):
    return lax.dot_general(x, w, (((x.ndim - 1,), (0,)), ((), ())),
                           precision=_PREC, preferred_element_type=jnp.float32)


def _sc_edge_kernel(db_hbm, eh_hbm, ce_hbm, sadj_hbm, dadj_hbm, dpl_hbm,
                    e_new_hbm, nd_hbm, stats_hbm,
                    sidx_v, didx_v, dpl_v,
                    db_v, eh_v, ce_v, en_v, nd_v, zb_v, stat_v,
                    nd_sh):
    c = lax.axis_index("c")
    s = lax.axis_index("s")
    zero16 = jnp.zeros((16,), jnp.float32)

    def _zb(i, _):
        for j in range(8):
            zb_v[i, pl.ds(j * 16, 16)] = zero16
        return 0
    lax.fori_loop(0, 16, _zb, 0)
    zrow = s * ROWS_PER_TILE
    def _zinit(k, _):
        pltpu.sync_copy(zb_v, nd_sh.at[pl.ds(zrow + k * 16, 16)])
        return 0
    lax.fori_loop(0, ROWS_PER_TILE // 16, _zinit, 0)
    pltpu.sync_copy(zb_v.at[pl.ds(0, ROWS_PER_TILE % 16)],
                    nd_sh.at[pl.ds(zrow + (ROWS_PER_TILE // 16) * 16,
                                   ROWS_PER_TILE % 16)])
    plsc.subcore_barrier()

    ebase = s * EPW
    irow = s * NCH

    def _block(B, carry):
        rb = irow + B * IDXB
        pltpu.sync_copy(sadj_hbm.at[c, pl.ds(rb, IDXB)], sidx_v)
        pltpu.sync_copy(dadj_hbm.at[c, pl.ds(rb, IDXB)], didx_v)
        pltpu.sync_copy(dpl_hbm.at[pl.ds(rb, IDXB)], dpl_v)

        def _chunk(j, cy):
            b = 0
            k = B * IDXB + j
            off = ebase + k * CH
            pltpu.sync_copy(db_hbm.at[sidx_v.at[j]], db_v.at[b])
            pltpu.sync_copy(eh_hbm.at[didx_v.at[j]], eh_v.at[b])
            pltpu.sync_copy(ce_hbm.at[c, pl.ds(off, CH)], ce_v.at[b])

            def _row(r, cy2):
                cy2 = list(cy2)
                for q in range(4):
                    sl = pl.ds(q * 16, 16)
                    d = db_v[b, r, sl]
                    bb = db_v[b, r, pl.ds(64 + q * 16, 16)]
                    x = d + eh_v[b, r, sl] + ce_v[b, r, sl]
                    en_v[b, r, sl] = x
                    cy2[q] = cy2[q] + x
                    cy2[4 + q] = cy2[4 + q] + x * x
                    sg = 1.0 / (1.0 + jnp.exp(-x))
                    nd_v[b, r, pl.ds(64 + q * 16, 16)] = sg
                    nd_v[b, r, sl] = sg * bb
                return tuple(cy2)
            cy = lax.fori_loop(0, CH, _row, cy)

            pltpu.sync_copy(en_v.at[b], e_new_hbm.at[c, pl.ds(off, CH)])
            pltpu.sync_copy(nd_v.at[b], nd_sh.at[dpl_v.at[j]], add=True)
            return cy
        return lax.fori_loop(0, IDXB, _chunk, carry)

    carry0 = tuple(zero16 for _ in range(8))
    carry = lax.fori_loop(0, NBLK, _block, carry0)

    for j in range(4):
        stat_v[0, pl.ds(j * 16, 16)] = carry[j]
        stat_v[1, pl.ds(j * 16, 16)] = carry[4 + j]
    pltpu.sync_copy(stat_v, stats_hbm.at[c, s])

    plsc.subcore_barrier()
    rbase = s * ROWS_PER_TILE
    pltpu.sync_copy(nd_sh.at[pl.ds(rbase, ROWS_PER_TILE)],
                    nd_hbm.at[c, pl.ds(rbase, ROWS_PER_TILE)])


def _sc_edge_pass(db_tab, eh_tab, ce, sadj, dadj, dpl):
    mesh = plsc.VectorSubcoreMesh(core_axis_name="c", subcore_axis_name="s")
    f32 = jnp.float32
    i32 = jnp.int32
    out_type = (
        jax.ShapeDtypeStruct((NC, E, HC), f32),
        jax.ShapeDtypeStruct((NC, NP, H), f32),
        jax.ShapeDtypeStruct((NC, NS, 2, HC), f32),
    )
    scratch = [
        pltpu.VMEM((IDXB, CH), i32), pltpu.VMEM((IDXB, CH), i32),
        pltpu.VMEM((IDXB, CH), i32),
        pltpu.VMEM((2, CH, H), f32), pltpu.VMEM((2, CH, H), f32),
        pltpu.VMEM((2, CH, HC), f32),
        pltpu.VMEM((2, CH, HC), f32), pltpu.VMEM((2, CH, H), f32),
        pltpu.VMEM((16, H), f32), pltpu.VMEM((2, HC), f32),
        pltpu.VMEM_SHARED((NP, H), f32),
    ]
    fn = pl.kernel(_sc_edge_kernel, out_type=out_type, mesh=mesh,
                   scratch_types=scratch,
                   compiler_params=pltpu.CompilerParams(
                       use_tc_tiling_on_sc=False))
    return fn(db_tab, eh_tab, ce, sadj, dadj, dpl)


BE = 2000


def _embed_h_kernel(h_ref, wh_ref, bh_ref, out_ref):
    out_ref[...] = _dot(h_ref[...], wh_ref[...]) + bh_ref[...]


def _embed_e_kernel(e_ref, we_ref, be_ref, wc_ref, bc_ref, emb_ref, ce_ref):
    x = _dot(e_ref[...], we_ref[...]) + be_ref[...]
    emb_ref[0] = x[:, :HC]
    emb_ref[1] = x[:, HC:]
    cn = _dot(x, wc_ref[...]) + bc_ref[...]
    ce_ref[0] = cn[:, :HC]
    ce_ref[1] = cn[:, HC:]


def _node_mm_kernel(h_ref, wa_ref, ba_ref, wdb_ref, bdb_ref, weh_ref, beh_ref,
                    ah_ref, db_ref, eh_ref):
    hh = h_ref[...]
    ah_ref[...] = _dot(hh, wa_ref[...]) + ba_ref[...]
    for cc in range(NC):
        db_ref[cc] = _dot(hh, wdb_ref[cc]) + bdb_ref[cc]
        eh_ref[cc] = _dot(hh, weh_ref[cc]) + beh_ref[cc]


def _node_update_kernel(ah_ref, nd_ref, hin_ref, stats_ref,
                        gh_ref, bh_ref, ge_ref, be_ref,
                        hout_ref, scale_ref, shift_ref):
    num = jnp.concatenate([nd_ref[0, :N, :HC], nd_ref[1, :N, :HC]], axis=1)
    den = jnp.concatenate([nd_ref[0, :N, HC:], nd_ref[1, :N, HC:]], axis=1)
    hn = ah_ref[...] + num / (den + 1e-6)
    mu = jnp.mean(hn, axis=0, keepdims=True)
    var = jnp.mean((hn - mu) ** 2, axis=0, keepdims=True)
    hn = gh_ref[...] * (hn - mu) * lax.rsqrt(var + 1e-5) + bh_ref[...]
    hout_ref[...] = hin_ref[...] + jnp.maximum(hn, 0.0)
    st = stats_ref[...]
    ssum = jnp.sum(st[:, :, 0, :], axis=1)
    ssq = jnp.sum(st[:, :, 1, :], axis=1)
    mu_e = ssum / E
    var_e = ssq / E - mu_e * mu_e
    sc = ge_ref[...] * lax.rsqrt(var_e + 1e-5)
    scale_ref[...] = sc
    shift_ref[...] = be_ref[...] - sc * mu_e


def _edge_update_kernel(en_ref, ein_ref, scale_ref, shift_ref, wn_ref, bn_ref,
                        eout_ref, cen_ref):
    x = en_ref[...]
    scl = scale_ref[...]
    sft = shift_ref[...]
    t = jnp.maximum(x * scl[:, None, :] + sft[:, None, :], 0.0)
    eo = ein_ref[...] + t
    eout_ref[...] = eo
    row = jnp.concatenate([eo[0], eo[1]], axis=1)
    cn = _dot(row, wn_ref[...]) + bn_ref[...]
    cen_ref[0] = cn[:, :HC]
    cen_ref[1] = cn[:, HC:]


def _readout_kernel(h_ref, m0_ref, b0_ref, m1_ref, b1_ref, m2_ref, b2_ref,
                    y_ref):
    y = jnp.maximum(_dot(h_ref[...], m0_ref[...]) + b0_ref[...], 0.0)
    y = jnp.maximum(_dot(y, m1_ref[...]) + b1_ref[...], 0.0)
    y_ref[...] = _dot(y, m2_ref[...]) + b2_ref[...]


def kernel(h, e, edge_index, W_h, b_h, W_e, b_e, WA, bA, WB, bB, WC, bC,
           WD, bD, WE, bE, gamma_h, beta_h, gamma_e, beta_e,
           M0, m0, M1, m1, M2, m2):
    f32 = jnp.float32
    src = edge_index[0]
    dst = edge_index[1]
    sadj = jnp.stack([src.reshape(ROWC, CH), (src + N).reshape(ROWC, CH)])
    dadj = jnp.stack([dst.reshape(ROWC, CH), (dst + N).reshape(ROWC, CH)])
    dpl = dst.reshape(ROWC, CH)

    W_db = jnp.stack([
        jnp.stack([jnp.concatenate([WD[l, :, c * HC:(c + 1) * HC],
                                    WB[l, :, c * HC:(c + 1) * HC]], axis=1)
                   for c in range(NC)])
        for l in range(L)])
    b_db = jnp.stack([
        jnp.stack([jnp.concatenate([bD[l, c * HC:(c + 1) * HC],
                                    bB[l, c * HC:(c + 1) * HC]])[None]
                   for c in range(NC)])
        for l in range(L)])
    zpad = jnp.zeros((H, HC), f32)
    W_eh = jnp.stack([
        jnp.stack([jnp.concatenate([WE[l, :, c * HC:(c + 1) * HC], zpad],
                                   axis=1) for c in range(NC)])
        for l in range(L)])
    b_eh = jnp.stack([
        jnp.stack([jnp.concatenate([bE[l, c * HC:(c + 1) * HC],
                                    jnp.zeros((HC,), f32)])[None]
                   for c in range(NC)])
        for l in range(L)])

    h0 = pl.pallas_call(
        _embed_h_kernel,
        out_shape=jax.ShapeDtypeStruct((N, H), f32),
    )(h, W_h, b_h[None])

    grid_e = (E // BE,)
    emb_spec = pl.BlockSpec((NC, BE, HC), lambda i: (0, i, 0))
    e_blk = pl.BlockSpec((BE, 16), lambda i: (i, 0))
    full2 = lambda a: pl.BlockSpec(a.shape, lambda i: tuple(0 for _ in a.shape))
    e_lay, ce = pl.pallas_call(
        _embed_e_kernel,
        grid=grid_e,
        in_specs=[e_blk, full2(W_e), full2(b_e[None]), full2(WC[0]),
                  full2(bC[0][None])],
        out_specs=[emb_spec, emb_spec],
        out_shape=[jax.ShapeDtypeStruct((NC, E, HC), f32),
                   jax.ShapeDtypeStruct((NC, E, HC), f32)],
    )(e, W_e, b_e[None], WC[0], bC[0][None])

    hcur = h0
    for l in range(L):
        BN_ = 2000
        nblk = pl.BlockSpec((BN_, H), lambda i: (i, 0))
        nblk2 = pl.BlockSpec((NC, BN_, H), lambda i: (0, i, 0))
        ah, db_tab, eh_tab = pl.pallas_call(
            _node_mm_kernel,
            grid=(N // BN_,),
            in_specs=[nblk, full2(WA[l]), full2(bA[l][None]),
                      full2(W_db[l]), full2(b_db[l]),
                      full2(W_eh[l]), full2(b_eh[l])],
            out_specs=[nblk, nblk2, nblk2],
            out_shape=[jax.ShapeDtypeStruct((N, H), f32),
                       jax.ShapeDtypeStruct((NC, N, H), f32),
                       jax.ShapeDtypeStruct((NC, N, H), f32)],
        )(hcur, WA[l], bA[l][None], W_db[l], b_db[l], W_eh[l], b_eh[l])

        e_new, nd, stats = _sc_edge_pass(
            db_tab.reshape(NC * N, H), eh_tab.reshape(NC * N, H),
            ce, sadj, dadj, dpl)

        hcur, scale, shift = pl.pallas_call(
            _node_update_kernel,
            out_shape=[jax.ShapeDtypeStruct((N, H), f32),
                       jax.ShapeDtypeStruct((NC, HC), f32),
                       jax.ShapeDtypeStruct((NC, HC), f32)],
        )(ah, nd, hcur, stats, gamma_h[l][None], beta_h[l][None],
          gamma_e[l].reshape(NC, HC), beta_e[l].reshape(NC, HC))

        if l < L - 1:
            wn, bn = WC[l + 1], bC[l + 1]
            blk = pl.BlockSpec((NC, BE, HC), lambda i: (0, i, 0))
            e_lay, ce = pl.pallas_call(
                _edge_update_kernel,
                grid=grid_e,
                in_specs=[blk, blk, full2(scale), full2(shift), full2(wn),
                          full2(bn[None])],
                out_specs=[blk, blk],
                out_shape=[jax.ShapeDtypeStruct((NC, E, HC), f32),
                           jax.ShapeDtypeStruct((NC, E, HC), f32)],
            )(e_new, e_lay, scale, shift, wn, bn[None])

    y = pl.pallas_call(
        _readout_kernel,
        out_shape=jax.ShapeDtypeStruct((N, 10), f32),
    )(hcur, M0, m0[None], M1, m1[None], M2, m2[None])
    return y


# speedup vs baseline: 1.1416x; 1.1416x over previous
import functools

import jax
import jax.numpy as jnp
from jax import lax
from jax.experimental import pallas as pl
from jax.experimental.pallas import tpu as pltpu
from jax.experimental.pallas import tpu_sc as plsc

N = 10000
E = 320000
H = 128
HC = H // 2
L = 4
NC = 2
NS = 16
EPW = E // NS
CH = 80
NCH = EPW // CH
IDXB = 10
NBLK = NCH // IDXB
ROWC = E // CH
ROWS_PER_TILE = 632
NP = ROWS_PER_TILE * NS

_PREC = lax.Precision.DEFAULT


def _dot(x, w):
    return lax.dot_general(x, w, (((x.ndim - 1,), (0,)), ((), ())),
                           precision=_PREC, preferred_element_type=jnp.float32)


def _sc_edge_kernel(db_hbm, eh_hbm, ce_hbm, edg_hbm,
                    e_new_hbm, nd_hbm, stats_hbm,
                    idx_v, db_v, eh_v, ce_v, en_v, nd_v, zb_v, stat_v,
                    nd_sh):
    c = lax.axis_index("c")
    s = lax.axis_index("s")
    zero16 = jnp.zeros((16,), jnp.float32)

    def _zb(i, _):
        for j in range(8):
            zb_v[i, pl.ds(j * 16, 16)] = zero16
        return 0
    lax.fori_loop(0, 16, _zb, 0)
    zrow = s * ROWS_PER_TILE
    def _zinit(k, _):
        pltpu.sync_copy(zb_v, nd_sh.at[pl.ds(zrow + k * 16, 16)])
        return 0
    lax.fori_loop(0, ROWS_PER_TILE // 16, _zinit, 0)
    pltpu.sync_copy(zb_v.at[pl.ds(0, ROWS_PER_TILE % 16)],
                    nd_sh.at[pl.ds(zrow + (ROWS_PER_TILE // 16) * 16,
                                   ROWS_PER_TILE % 16)])
    plsc.subcore_barrier()

    ebase = s * EPW
    irow = s * NCH

    def _chunk_body(B, j, cy):
        k = B * IDXB + j
        off = ebase + k * CH
        pltpu.sync_copy(db_hbm.at[idx_v.at[j, 0]], db_v)
        pltpu.sync_copy(eh_hbm.at[idx_v.at[j, 1]], eh_v)
        pltpu.sync_copy(ce_hbm.at[c, pl.ds(off, CH)], ce_v)

        def _row(r, cy2):
            cy2 = list(cy2)
            for q in range(4):
                sl = pl.ds(q * 16, 16)
                d = db_v[r, sl]
                bb = db_v[r, pl.ds(64 + q * 16, 16)]
                x = d + eh_v[r, sl] + ce_v[r, sl]
                en_v[r, sl] = x
                cy2[q] = cy2[q] + x
                cy2[4 + q] = cy2[4 + q] + x * x
                sg = 1.0 / (1.0 + jnp.exp(-x))
                nd_v[r, pl.ds(64 + q * 16, 16)] = sg
                nd_v[r, sl] = sg * bb
            return tuple(cy2)
        cy = lax.fori_loop(0, CH, _row, cy)

        pltpu.sync_copy(en_v, e_new_hbm.at[c, pl.ds(off, CH)])
        pltpu.sync_copy(nd_v, nd_sh.at[idx_v.at[j, 2]], add=True)
        return cy

    def _block(B, carry):
        pltpu.sync_copy(edg_hbm.at[c, pl.ds(irow + B * IDXB, IDXB)], idx_v)
        return lax.fori_loop(0, IDXB,
                             lambda j, cy: _chunk_body(B, j, cy), carry)

    carry0 = tuple(zero16 for _ in range(8))
    carry = lax.fori_loop(0, NBLK, _block, carry0)

    for j in range(4):
        stat_v[0, pl.ds(j * 16, 16)] = carry[j]
        stat_v[1, pl.ds(j * 16, 16)] = carry[4 + j]
    pltpu.sync_copy(stat_v, stats_hbm.at[c, s])

    plsc.subcore_barrier()
    rbase = s * ROWS_PER_TILE
    pltpu.sync_copy(nd_sh.at[pl.ds(rbase, ROWS_PER_TILE)],
                    nd_hbm.at[c, pl.ds(rbase, ROWS_PER_TILE)])


def _sc_edge_pass(db_tab, eh_tab, ce, edg):
    mesh = plsc.VectorSubcoreMesh(core_axis_name="c", subcore_axis_name="s")
    f32 = jnp.float32
    i32 = jnp.int32
    out_type = (
        jax.ShapeDtypeStruct((NC, E, HC), f32),
        jax.ShapeDtypeStruct((NC, NP, H), f32),
        jax.ShapeDtypeStruct((NC, NS, 2, HC), f32),
    )
    scratch = [
        pltpu.VMEM((IDXB, 3, CH), i32),
        pltpu.VMEM((CH, H), f32), pltpu.VMEM((CH, H), f32),
        pltpu.VMEM((CH, HC), f32),
        pltpu.VMEM((CH, HC), f32), pltpu.VMEM((CH, H), f32),
        pltpu.VMEM((16, H), f32), pltpu.VMEM((2, HC), f32),
        pltpu.VMEM_SHARED((NP, H), f32),
    ]
    fn = pl.kernel(_sc_edge_kernel, out_type=out_type, mesh=mesh,
                   scratch_types=scratch,
                   compiler_params=pltpu.CompilerParams(
                       use_tc_tiling_on_sc=False))
    return fn(db_tab, eh_tab, ce, edg)


BE = 2000


def _embed_h_kernel(h_ref, wh_ref, bh_ref, out_ref):
    out_ref[...] = _dot(h_ref[...], wh_ref[...]) + bh_ref[...]


def _embed_e_kernel(e_ref, we_ref, be_ref, wc_ref, bc_ref, emb_ref, ce_ref):
    x = _dot(e_ref[...], we_ref[...]) + be_ref[...]
    emb_ref[0] = x[:, :HC]
    emb_ref[1] = x[:, HC:]
    cn = _dot(x, wc_ref[...]) + bc_ref[...]
    ce_ref[0] = cn[:, :HC]
    ce_ref[1] = cn[:, HC:]


def _node_mm_kernel(h_ref, wa_ref, ba_ref, wdb_ref, bdb_ref, weh_ref, beh_ref,
                    ah_ref, db_ref, eh_ref):
    hh = h_ref[...]
    ah_ref[...] = _dot(hh, wa_ref[...]) + ba_ref[...]
    for cc in range(NC):
        db_ref[cc] = _dot(hh, wdb_ref[cc]) + bdb_ref[cc]
        eh_ref[cc] = _dot(hh, weh_ref[cc]) + beh_ref[cc]


def _node_update_kernel(ah_ref, nd_ref, hin_ref, stats_ref,
                        gh_ref, bh_ref, ge_ref, be_ref,
                        hout_ref, scale_ref, shift_ref):
    num = jnp.concatenate([nd_ref[0, :N, :HC], nd_ref[1, :N, :HC]], axis=1)
    den = jnp.concatenate([nd_ref[0, :N, HC:], nd_ref[1, :N, HC:]], axis=1)
    hn = ah_ref[...] + num / (den + 1e-6)
    mu = jnp.mean(hn, axis=0, keepdims=True)
    var = jnp.mean((hn - mu) ** 2, axis=0, keepdims=True)
    hn = gh_ref[...] * (hn - mu) * lax.rsqrt(var + 1e-5) + bh_ref[...]
    hout_ref[...] = hin_ref[...] + jnp.maximum(hn, 0.0)
    st = stats_ref[...]
    ssum = jnp.sum(st[:, :, 0, :], axis=1)
    ssq = jnp.sum(st[:, :, 1, :], axis=1)
    mu_e = ssum / E
    var_e = ssq / E - mu_e * mu_e
    sc = ge_ref[...] * lax.rsqrt(var_e + 1e-5)
    scale_ref[...] = sc
    shift_ref[...] = be_ref[...] - sc * mu_e


def _edge_update_kernel(en_ref, ein_ref, scale_ref, shift_ref, wn_ref, bn_ref,
                        eout_ref, cen_ref):
    x = en_ref[...]
    scl = scale_ref[...]
    sft = shift_ref[...]
    t = jnp.maximum(x * scl[:, None, :] + sft[:, None, :], 0.0)
    eo = ein_ref[...] + t
    eout_ref[...] = eo
    row = jnp.concatenate([eo[0], eo[1]], axis=1)
    cn = _dot(row, wn_ref[...]) + bn_ref[...]
    cen_ref[0] = cn[:, :HC]
    cen_ref[1] = cn[:, HC:]


def _readout_kernel(h_ref, m0_ref, b0_ref, m1_ref, b1_ref, m2_ref, b2_ref,
                    y_ref):
    y = jnp.maximum(_dot(h_ref[...], m0_ref[...]) + b0_ref[...], 0.0)
    y = jnp.maximum(_dot(y, m1_ref[...]) + b1_ref[...], 0.0)
    y_ref[...] = _dot(y, m2_ref[...]) + b2_ref[...]


def kernel(h, e, edge_index, W_h, b_h, W_e, b_e, WA, bA, WB, bB, WC, bC,
           WD, bD, WE, bE, gamma_h, beta_h, gamma_e, beta_e,
           M0, m0, M1, m1, M2, m2):
    f32 = jnp.float32
    src = edge_index[0]
    dst = edge_index[1]
    edg = jnp.stack([
        jnp.stack([(src + cc * N).reshape(ROWC, CH),
                   (dst + cc * N).reshape(ROWC, CH),
                   dst.reshape(ROWC, CH)], axis=1)
        for cc in range(NC)])
    edg = jnp.pad(edg, ((0, 0), (0, IDXB), (0, 0), (0, 0)))

    W_db = jnp.stack([
        jnp.stack([jnp.concatenate([WD[l, :, c * HC:(c + 1) * HC],
                                    WB[l, :, c * HC:(c + 1) * HC]], axis=1)
                   for c in range(NC)])
        for l in range(L)])
    b_db = jnp.stack([
        jnp.stack([jnp.concatenate([bD[l, c * HC:(c + 1) * HC],
                                    bB[l, c * HC:(c + 1) * HC]])[None]
                   for c in range(NC)])
        for l in range(L)])
    zpad = jnp.zeros((H, HC), f32)
    W_eh = jnp.stack([
        jnp.stack([jnp.concatenate([WE[l, :, c * HC:(c + 1) * HC], zpad],
                                   axis=1) for c in range(NC)])
        for l in range(L)])
    b_eh = jnp.stack([
        jnp.stack([jnp.concatenate([bE[l, c * HC:(c + 1) * HC],
                                    jnp.zeros((HC,), f32)])[None]
                   for c in range(NC)])
        for l in range(L)])

    h0 = pl.pallas_call(
        _embed_h_kernel,
        out_shape=jax.ShapeDtypeStruct((N, H), f32),
    )(h, W_h, b_h[None])

    grid_e = (E // BE,)
    emb_spec = pl.BlockSpec((NC, BE, HC), lambda i: (0, i, 0))
    e_blk = pl.BlockSpec((BE, 16), lambda i: (i, 0))
    full2 = lambda a: pl.BlockSpec(a.shape, lambda i: tuple(0 for _ in a.shape))
    e_lay, ce = pl.pallas_call(
        _embed_e_kernel,
        grid=grid_e,
        in_specs=[e_blk, full2(W_e), full2(b_e[None]), full2(WC[0]),
                  full2(bC[0][None])],
        out_specs=[emb_spec, emb_spec],
        out_shape=[jax.ShapeDtypeStruct((NC, E, HC), f32),
                   jax.ShapeDtypeStruct((NC, E, HC), f32)],
    )(e, W_e, b_e[None], WC[0], bC[0][None])

    hcur = h0
    for l in range(L):
        BN_ = 2000
        nblk = pl.BlockSpec((BN_, H), lambda i: (i, 0))
        nblk2 = pl.BlockSpec((NC, BN_, H), lambda i: (0, i, 0))
        ah, db_tab, eh_tab = pl.pallas_call(
            _node_mm_kernel,
            grid=(N // BN_,),
            in_specs=[nblk, full2(WA[l]), full2(bA[l][None]),
                      full2(W_db[l]), full2(b_db[l]),
                      full2(W_eh[l]), full2(b_eh[l])],
            out_specs=[nblk, nblk2, nblk2],
            out_shape=[jax.ShapeDtypeStruct((N, H), f32),
                       jax.ShapeDtypeStruct((NC, N, H), f32),
                       jax.ShapeDtypeStruct((NC, N, H), f32)],
        )(hcur, WA[l], bA[l][None], W_db[l], b_db[l], W_eh[l], b_eh[l])

        e_new, nd, stats = _sc_edge_pass(
            db_tab.reshape(NC * N, H), eh_tab.reshape(NC * N, H),
            ce, edg)

        hcur, scale, shift = pl.pallas_call(
            _node_update_kernel,
            out_shape=[jax.ShapeDtypeStruct((N, H), f32),
                       jax.ShapeDtypeStruct((NC, HC), f32),
                       jax.ShapeDtypeStruct((NC, HC), f32)],
        )(ah, nd, hcur, stats, gamma_h[l][None], beta_h[l][None],
          gamma_e[l].reshape(NC, HC), beta_e[l].reshape(NC, HC))

        if l < L - 1:
            wn, bn = WC[l + 1], bC[l + 1]
            blk = pl.BlockSpec((NC, BE, HC), lambda i: (0, i, 0))
            e_lay, ce = pl.pallas_call(
                _edge_update_kernel,
                grid=grid_e,
                in_specs=[blk, blk, full2(scale), full2(shift), full2(wn),
                          full2(bn[None])],
                out_specs=[blk, blk],
                out_shape=[jax.ShapeDtypeStruct((NC, E, HC), f32),
                           jax.ShapeDtypeStruct((NC, E, HC), f32)],
            )(e_new, e_lay, scale, shift, wn, bn[None])

    y = pl.pallas_call(
        _readout_kernel,
        out_shape=jax.ShapeDtypeStruct((N, 10), f32),
    )(hcur, M0, m0[None], M1, m1[None], M2, m2[None])
    return y
